# baseline (speedup 1.0000x reference)
import jax
import jax.numpy as jnp
from jax import lax
from jax.experimental import pallas as pl

_NB = 8


def _gn_silu_rows(y, nb, S, g_ref, bt_ref):
    G = (lax.broadcasted_iota(jnp.int32, (128, 32), 0) // 4 ==
         lax.broadcasted_iota(jnp.int32, (128, 32), 1)).astype(jnp.float32)
    GT = (lax.broadcasted_iota(jnp.int32, (32, 128), 0) ==
          lax.broadcasted_iota(jnp.int32, (32, 128), 1) // 4
          ).astype(jnp.float32)
    cnt = jnp.float32(4 * S)
    mu = jnp.sum(jnp.dot(y, G, preferred_element_type=jnp.float32)
                 .reshape(nb, S, 32), axis=1) / cnt
    mu128 = jnp.dot(mu, GT, preferred_element_type=jnp.float32)
    yv = y.reshape(nb, S, 128)
    dev = yv - mu128[:, None, :]
    var = jnp.sum(jnp.dot((dev * dev).reshape(nb * S, 128), G,
                          preferred_element_type=jnp.float32)
                  .reshape(nb, S, 32), axis=1) / cnt
    v128 = jnp.dot(var, GT, preferred_element_type=jnp.float32)
    xn = dev / jnp.sqrt(v128 + 1e-5)[:, None, :]
    out = xn * g_ref[...].reshape(1, 1, 128) + bt_ref[...].reshape(1, 1, 128)
    return jax.nn.silu(out).reshape(nb * S, 128)


def _take3(a, axis, d):
    start = 0 if d == 0 else 1
    sl = lax.slice_in_dim(a, start, start + 6, axis=axis)
    shp = list(sl.shape)
    shp[axis:axis + 1] = [3, 2]
    sl = sl.reshape(shp)
    parity = 0 if d < 2 else 1
    return lax.index_in_dim(sl, parity, axis=axis + 1, keepdims=False)


def _mapper_from_patches(P, W1T_ref, b1_ref, g1_ref, bt1_ref, W2T_ref,
                         b2_ref, g2_ref, bt2_ref, W3T_ref, b3_ref, g3_ref,
                         bt3_ref, QWT_ref, Qb_ref):
    nb = P.shape[0]
    y1 = jnp.dot(P.reshape(nb * 343, 27), W1T_ref[...],
                 preferred_element_type=jnp.float32) + b1_ref[...]
    y1 = _gn_silu_rows(y1, nb, 343, g1_ref, bt1_ref)
    y2 = jnp.dot(y1, W2T_ref[...],
                 preferred_element_type=jnp.float32) + b2_ref[...]
    y2 = _gn_silu_rows(y2, nb, 343, g2_ref, bt2_ref)
    y2v = y2.reshape(nb, 7, 7, 7, 128)
    acc = jnp.zeros((nb * 27, 128), jnp.float32)
    d = 0
    for dz in range(3):
        tz = _take3(y2v, 1, dz)
        for dy in range(3):
            ty = _take3(tz, 2, dy)
            for dx in range(3):
                t = _take3(ty, 3, dx)
                acc = acc + jnp.dot(t.reshape(nb * 27, 128), W3T_ref[d],
                                    preferred_element_type=jnp.float32)
                d += 1
    y3 = acc + b3_ref[...]
    y3 = _gn_silu_rows(y3, nb, 27, g3_ref, bt3_ref)
    feats = jnp.mean(y3.reshape(nb, 27, 128), axis=1)
    return jnp.dot(feats, QWT_ref[...],
                   preferred_element_type=jnp.float32) + Qb_ref[...]


def _mapper_body(P_ref, W1T_ref, b1_ref, g1_ref, bt1_ref, W2T_ref, b2_ref,
                 g2_ref, bt2_ref, W3T_ref, b3_ref, g3_ref, bt3_ref,
                 QWT_ref, Qb_ref, o_ref):
    o_ref[...] = _mapper_from_patches(
        P_ref[...], W1T_ref, b1_ref, g1_ref, bt1_ref, W2T_ref, b2_ref,
        g2_ref, bt2_ref, W3T_ref, b3_ref, g3_ref, bt3_ref, QWT_ref, Qb_ref)


def _pair_body(Pa_ref, Pb_ref, W1T_ref, b1_ref, g1_ref, bt1_ref, W2T_ref,
               b2_ref, g2_ref, bt2_ref, W3T_ref, b3_ref, g3_ref, bt3_ref,
               QWT_ref, Qb_ref, o_ref):
    pair = Pa_ref[...].reshape(1, 343, 27) + Pb_ref[...]
    pair = pair - jax.nn.relu(pair - 1.0)
    o_ref[...] = _mapper_from_patches(
        pair, W1T_ref, b1_ref, g1_ref, bt1_ref, W2T_ref, b2_ref, g2_ref,
        bt2_ref, W3T_ref, b3_ref, g3_ref, bt3_ref, QWT_ref, Qb_ref)


def _glue_body(q1_ref, q2_ref, Pa_ref, Pb_ref, m_ref, pr_ref, gr_ref):
    q1 = q1_ref[...]
    Pa = Pa_ref[...]
    d1 = jnp.sqrt(jnp.maximum(
        jnp.sum((q1[:, None, :] - Pa[None, :, :]) ** 2, axis=-1), 1e-12))
    p1 = jnp.exp(-d1)
    pr_ref[...] = jnp.dot(p1, Pa, preferred_element_type=jnp.float32)
    q2 = q2_ref[...]
    Pb = Pb_ref[...]
    d2 = jnp.sqrt(jnp.maximum(
        jnp.sum((q2[:, None, :] - Pb[None, :, :]) ** 2, axis=-1), 1e-12))
    p2 = jnp.exp(-d2)
    K2 = (lax.broadcasted_iota(jnp.int32, (12, 6), 0) // 2 ==
          lax.broadcasted_iota(jnp.int32, (12, 6), 1)).astype(jnp.float32)
    pk = jnp.dot(p2, K2, preferred_element_type=jnp.float32)
    pred2 = pk / jnp.sum(pk, axis=1, keepdims=True)
    p2m = jnp.max(pred2, axis=1).reshape(16, 8)
    mm = m_ref[...]
    gr0 = jnp.dot(p2m[0:8, :], mm[0:8, :], preferred_element_type=jnp.float32)
    gr1 = jnp.dot(p2m[8:16, :], mm[8:16, :],
                  preferred_element_type=jnp.float32)
    gr_ref[...] = jnp.concatenate([gr0, gr1], axis=0)


_W_SHAPES = ((27, 128), (1, 128), (1, 128), (1, 128), (128, 128), (1, 128),
             (1, 128), (1, 128), (27, 128, 128), (1, 128), (1, 128), (1, 128),
             (128, 256), (1, 256))


def _full_spec(shape):
    return pl.BlockSpec(shape, lambda *args: (0,) * len(shape))


def _im2col(xflat):
    nv = xflat.shape[0]
    x = xflat.reshape(nv, 16, 16, 16)
    cols = []
    for dz in range(3):
        for dy in range(3):
            for dx in range(3):
                cols.append(lax.slice(
                    x, (0, dz, dy, dx), (nv, dz + 13, dy + 13, dx + 13),
                    (1, 2, 2, 2)))
    return jnp.stack(cols, axis=-1).reshape(nv, 343, 27)


def _run_mapper(Pm3, wargs):
    nv = Pm3.shape[0]
    return pl.pallas_call(
        _mapper_body,
        grid=(nv // _NB,),
        in_specs=[pl.BlockSpec((_NB, 343, 27), lambda i: (i, 0, 0))] +
                 [_full_spec(s) for s in _W_SHAPES],
        out_specs=pl.BlockSpec((_NB, 256), lambda i: (i, 0)),
        out_shape=jax.ShapeDtypeStruct((nv, 256), jnp.float32),
    )(Pm3, *wargs)


def _run_pair_mapper(Pm3, wargs):
    return pl.pallas_call(
        _pair_body,
        grid=(2, 8),
        in_specs=[pl.BlockSpec((1, 343, 27), lambda b, i: (b * 8 + i, 0, 0)),
                  pl.BlockSpec((8, 343, 27), lambda b, i: (b, 0, 0))] +
                 [pl.BlockSpec(s, (lambda b, i, _n=len(s): (0,) * _n))
                  for s in _W_SHAPES],
        out_specs=pl.BlockSpec((8, 256), lambda b, i: (b * 8 + i, 0)),
        out_shape=jax.ShapeDtypeStruct((128, 256), jnp.float32),
    )(Pm3, Pm3, *wargs)


def _run_glue(q1, q2, Pa, Pb, mflat):
    return pl.pallas_call(
        _glue_body,
        in_specs=[_full_spec((16, 256)), _full_spec((128, 256)),
                  _full_spec((12, 256)), _full_spec((12, 256)),
                  _full_spec((16, 4096))],
        out_specs=[_full_spec((16, 256)), _full_spec((16, 4096))],
        out_shape=[jax.ShapeDtypeStruct((16, 256), jnp.float32),
                   jax.ShapeDtypeStruct((16, 4096), jnp.float32)],
    )(q1, q2, Pa, Pb, mflat)


def kernel(m, W1, b1, g1, bt1, W2, b2, g2, bt2, W3, b3, g3, bt3,
           Q1_W, Q1_b, Q2_W, Q2_b, QH1_W, QH1_b, QH2_W, QH2_b,
           P1, P2, PH1, PH2):
    mflat = m.reshape(16, 4096)
    shared = (W1.reshape(128, 27).T, b1.reshape(1, 128), g1.reshape(1, 128),
              bt1.reshape(1, 128), W2.reshape(128, 128).T, b2.reshape(1, 128),
              g2.reshape(1, 128), bt2.reshape(1, 128),
              W3.reshape(128, 128, 27).transpose(2, 1, 0), b3.reshape(1, 128),
              g3.reshape(1, 128), bt3.reshape(1, 128))
    w1 = shared + (Q1_W.T, Q1_b.reshape(1, 256))
    w2 = shared + (Q2_W.T, Q2_b.reshape(1, 256))
    wh1 = shared + (QH1_W.T, QH1_b.reshape(1, 256))
    wh2 = shared + (QH2_W.T, QH2_b.reshape(1, 256))

    Pm3 = _im2col(mflat)
    q1 = _run_mapper(Pm3, w1)
    q2 = _run_pair_mapper(Pm3, w2)
    pr, gr = _run_glue(q1, q2, P1, P2, mflat)
    Ph3 = _im2col(gr)
    h1 = _run_mapper(Ph3, wh1)
    h2 = _run_pair_mapper(Ph3, wh2)
    phr, ghr = _run_glue(h1, h2, PH1, PH2, gr)
    return (pr.reshape(2, 8, 256), gr.reshape(2, 8, 1, 16, 16, 16),
            phr.reshape(2, 8, 256), ghr.reshape(2, 8, 1, 16, 16, 16))


# speedup vs baseline: 2.1880x; 2.1880x over previous
import jax
import jax.numpy as jnp
from jax import lax
from jax.experimental import pallas as pl


def _gn_silu(y, S, nb, g_ref, bt_ref):
    A4 = (lax.broadcasted_iota(jnp.int32, (128, 128), 0) // 4 ==
          lax.broadcasted_iota(jnp.int32, (128, 128), 1) // 4
          ).astype(jnp.float32) * jnp.float32(1.0 / (4 * S))
    yv = y.reshape(S, nb, 128)
    mu = jnp.dot(jnp.sum(yv, axis=0), A4,
                 preferred_element_type=jnp.float32)
    dev = yv - mu[None]
    v = jnp.dot(jnp.sum(dev * dev, axis=0), A4,
                preferred_element_type=jnp.float32)
    inv = 1.0 / jnp.sqrt(v + 1e-5)
    out = (dev * inv[None]) * g_ref[...].reshape(1, 1, 128) \
        + bt_ref[...].reshape(1, 1, 128)
    return jax.nn.silu(out).reshape(S * nb, 128)


def _take3(a, axis, d):
    start = 0 if d == 0 else 1
    sl = lax.slice_in_dim(a, start, start + 6, axis=axis)
    shp = list(sl.shape)
    shp[axis:axis + 1] = [3, 2]
    sl = sl.reshape(shp)
    parity = 0 if d < 2 else 1
    return lax.index_in_dim(sl, parity, axis=axis + 1, keepdims=False)


def _mapper_from_patches(P, W1T_ref, b1_ref, g1_ref, bt1_ref, W2T_ref,
                         b2_ref, g2_ref, bt2_ref, W3T_ref, b3_ref, g3_ref,
                         bt3_ref, QWT_ref, Qb_ref):
    nb = P.shape[1]
    y1 = jnp.dot(P.reshape(343 * nb, 27), W1T_ref[...],
                 preferred_element_type=jnp.float32) + b1_ref[...]
    y1 = _gn_silu(y1, 343, nb, g1_ref, bt1_ref)
    y2 = jnp.dot(y1, W2T_ref[...],
                 preferred_element_type=jnp.float32) + b2_ref[...]
    y2 = _gn_silu(y2, 343, nb, g2_ref, bt2_ref)
    y2v = y2.reshape(7, 7, 7, nb, 128)
    acc = jnp.zeros((27 * nb, 128), jnp.float32)
    d = 0
    for dz in range(3):
        tz = _take3(y2v, 0, dz)
        for dy in range(3):
            ty = _take3(tz, 1, dy)
            for dx in range(3):
                t = _take3(ty, 2, dx)
                acc = acc + jnp.dot(t.reshape(27 * nb, 128), W3T_ref[d],
                                    preferred_element_type=jnp.float32)
                d += 1
    y3 = _gn_silu(acc + b3_ref[...], 27, nb, g3_ref, bt3_ref)
    feats = jnp.mean(y3.reshape(27, nb, 128), axis=0)
    return jnp.dot(feats, QWT_ref[...],
                   preferred_element_type=jnp.float32) + Qb_ref[...]


def _mapper_body(P_ref, W1T_ref, b1_ref, g1_ref, bt1_ref, W2T_ref, b2_ref,
                 g2_ref, bt2_ref, W3T_ref, b3_ref, g3_ref, bt3_ref,
                 QWT_ref, Qb_ref, o_ref):
    o_ref[...] = _mapper_from_patches(
        P_ref[...], W1T_ref, b1_ref, g1_ref, bt1_ref, W2T_ref, b2_ref,
        g2_ref, bt2_ref, W3T_ref, b3_ref, g3_ref, bt3_ref, QWT_ref, Qb_ref)


def _pair_body(Pb_ref, W1T_ref, b1_ref, g1_ref, bt1_ref, W2T_ref, b2_ref,
               g2_ref, bt2_ref, W3T_ref, b3_ref, g3_ref, bt3_ref,
               QWT_ref, Qb_ref, o_ref):
    Pb = Pb_ref[...]
    pair = Pb[:, :, None, :] + Pb[:, None, :, :]
    pair = pair - jax.nn.relu(pair - 1.0)
    o_ref[...] = _mapper_from_patches(
        pair.reshape(343, 64, 27), W1T_ref, b1_ref, g1_ref, bt1_ref,
        W2T_ref, b2_ref, g2_ref, bt2_ref, W3T_ref, b3_ref, g3_ref, bt3_ref,
        QWT_ref, Qb_ref)


def _glue_body(q1_ref, q2_ref, Pa_ref, Pb_ref, m_ref, pr_ref, gr_ref):
    q1 = q1_ref[...]
    Pa = Pa_ref[...]
    d1 = jnp.sqrt(jnp.maximum(
        jnp.sum((q1[:, None, :] - Pa[None, :, :]) ** 2, axis=-1), 1e-12))
    p1 = jnp.exp(-d1)
    pr_ref[...] = jnp.dot(p1, Pa, preferred_element_type=jnp.float32)
    q2 = q2_ref[...]
    Pb = Pb_ref[...]
    d2 = jnp.sqrt(jnp.maximum(
        jnp.sum((q2[:, None, :] - Pb[None, :, :]) ** 2, axis=-1), 1e-12))
    p2 = jnp.exp(-d2)
    K2 = (lax.broadcasted_iota(jnp.int32, (12, 6), 0) // 2 ==
          lax.broadcasted_iota(jnp.int32, (12, 6), 1)).astype(jnp.float32)
    pk = jnp.dot(p2, K2, preferred_element_type=jnp.float32)
    pred2 = pk / jnp.sum(pk, axis=1, keepdims=True)
    p2m = jnp.max(pred2, axis=1).reshape(16, 8)
    mm = m_ref[...]
    gr0 = jnp.dot(p2m[0:8, :], mm[0:8, :], preferred_element_type=jnp.float32)
    gr1 = jnp.dot(p2m[8:16, :], mm[8:16, :],
                  preferred_element_type=jnp.float32)
    gr_ref[...] = jnp.concatenate([gr0, gr1], axis=0)


_W_SHAPES = ((27, 128), (1, 128), (1, 128), (1, 128), (128, 128), (1, 128),
             (1, 128), (1, 128), (27, 128, 128), (1, 128), (1, 128), (1, 128),
             (128, 256), (1, 256))


def _full_spec(shape):
    return pl.BlockSpec(shape, lambda *args: (0,) * len(shape))


def _im2col(xflat):
    nv = xflat.shape[0]
    x = xflat.reshape(nv, 16, 16, 16)
    cols = []
    for dz in range(3):
        for dy in range(3):
            for dx in range(3):
                cols.append(lax.slice(
                    x, (0, dz, dy, dx), (nv, dz + 13, dy + 13, dx + 13),
                    (1, 2, 2, 2)))
    return jnp.stack(cols, axis=-1).reshape(nv, 343, 27).transpose(1, 0, 2)


def _run_mapper(Pm3, wargs):
    nv = Pm3.shape[1]
    return pl.pallas_call(
        _mapper_body,
        grid=(1,),
        in_specs=[pl.BlockSpec((343, nv, 27), lambda i: (0, 0, 0))] +
                 [_full_spec(s) for s in _W_SHAPES],
        out_specs=pl.BlockSpec((nv, 256), lambda i: (0, 0)),
        out_shape=jax.ShapeDtypeStruct((nv, 256), jnp.float32),
    )(Pm3, *wargs)


def _run_pair_mapper(Pm3, wargs):
    return pl.pallas_call(
        _pair_body,
        grid=(2,),
        in_specs=[pl.BlockSpec((343, 8, 27), lambda b: (0, b, 0))] +
                 [pl.BlockSpec(s, (lambda b, _n=len(s): (0,) * _n))
                  for s in _W_SHAPES],
        out_specs=pl.BlockSpec((64, 256), lambda b: (b, 0)),
        out_shape=jax.ShapeDtypeStruct((128, 256), jnp.float32),
    )(Pm3, *wargs)


def _run_glue(q1, q2, Pa, Pb, mflat):
    return pl.pallas_call(
        _glue_body,
        in_specs=[_full_spec((16, 256)), _full_spec((128, 256)),
                  _full_spec((12, 256)), _full_spec((12, 256)),
                  _full_spec((16, 4096))],
        out_specs=[_full_spec((16, 256)), _full_spec((16, 4096))],
        out_shape=[jax.ShapeDtypeStruct((16, 256), jnp.float32),
                   jax.ShapeDtypeStruct((16, 4096), jnp.float32)],
    )(q1, q2, Pa, Pb, mflat)


def kernel(m, W1, b1, g1, bt1, W2, b2, g2, bt2, W3, b3, g3, bt3,
           Q1_W, Q1_b, Q2_W, Q2_b, QH1_W, QH1_b, QH2_W, QH2_b,
           P1, P2, PH1, PH2):
    mflat = m.reshape(16, 4096)
    shared = (W1.reshape(128, 27).T, b1.reshape(1, 128), g1.reshape(1, 128),
              bt1.reshape(1, 128), W2.reshape(128, 128).T, b2.reshape(1, 128),
              g2.reshape(1, 128), bt2.reshape(1, 128),
              W3.reshape(128, 128, 27).transpose(2, 1, 0), b3.reshape(1, 128),
              g3.reshape(1, 128), bt3.reshape(1, 128))
    w1 = shared + (Q1_W.T, Q1_b.reshape(1, 256))
    w2 = shared + (Q2_W.T, Q2_b.reshape(1, 256))
    wh1 = shared + (QH1_W.T, QH1_b.reshape(1, 256))
    wh2 = shared + (QH2_W.T, QH2_b.reshape(1, 256))

    Pm3 = _im2col(mflat)
    q1 = _run_mapper(Pm3, w1)
    q2 = _run_pair_mapper(Pm3, w2)
    pr, gr = _run_glue(q1, q2, P1, P2, mflat)
    Ph3 = _im2col(gr)
    h1 = _run_mapper(Ph3, wh1)
    h2 = _run_pair_mapper(Ph3, wh2)
    phr, ghr = _run_glue(h1, h2, PH1, PH2, gr)
    return (pr.reshape(2, 8, 256), gr.reshape(2, 8, 1, 16, 16, 16),
            phr.reshape(2, 8, 256), ghr.reshape(2, 8, 1, 16, 16, 16))
